# Initial kernel scaffold; baseline (speedup 1.0000x reference)
import functools

import jax
import jax.numpy as jnp
from jax import lax
from jax.experimental import pallas as pl
from jax.experimental.pallas import tpu as pltpu
from jax.experimental.pallas import tpu_sc as plsc

N = 10000
E = 320000
NG = 16
NC, NS = 2, 16
NW = NC * NS
CH = 128
CPT = 79
E_PAD = NW * CPT * CH
N_ACC = 10016
STRIPE = N_ACC // NS
DUMMY = 10008

_f32 = jnp.float32


def _sc_body(with_e, d_feat, src_r, tgt_r, *rest):
    if with_e:
        (e3_r, tab_r, zg_r, ze_r, g_out, ed_out,
         idx_s, idx_t, erows, hrows, sem, acc_g, acc_e) = rest
    else:
        (tab_r, zg_r, g_out,
         idx_s, idx_t, hrows, sem, acc_g) = rest

    c = lax.axis_index("c")
    s = lax.axis_index("s")
    wid = s * NC + c

    row0 = s * STRIPE
    pltpu.sync_copy(zg_r.at[pl.ds(row0, STRIPE)], acc_g.at[pl.ds(row0, STRIPE)])
    if with_e:
        pltpu.sync_copy(ze_r.at[pl.ds(row0, STRIPE)],
                        acc_e.at[pl.ds(row0, STRIPE)])

    pltpu.sync_copy(src_r.at[pl.ds(wid * CPT, CPT)], idx_s)
    pltpu.sync_copy(tgt_r.at[pl.ds(wid * CPT, CPT)], idx_t)
    plsc.subcore_barrier()

    def chunk(j, carry):
        if with_e:
            pltpu.sync_copy(e3_r.at[wid * CPT + j], erows)
        pltpu.async_copy(tab_r.at[idx_s.at[j]], hrows, sem).wait()
        if with_e:
            pltpu.sync_copy(erows, acc_e.at[idx_t.at[j]], add=True)
        pltpu.sync_copy(hrows, acc_g.at[idx_t.at[j]], add=True)
        return carry

    lax.fori_loop(0, CPT, chunk, 0)
    plsc.subcore_barrier()

    pltpu.sync_copy(acc_g.at[pl.ds(row0, STRIPE)],
                    g_out.at[c].at[pl.ds(row0, STRIPE)])
    if with_e:
        pltpu.sync_copy(acc_e.at[pl.ds(row0, STRIPE)],
                        ed_out.at[c].at[pl.ds(row0, STRIPE)])


def _make_sc(with_e, d_feat):
    mesh = plsc.VectorSubcoreMesh(core_axis_name="c", subcore_axis_name="s",
                                  num_cores=NC, num_subcores=NS)
    outs = [jax.ShapeDtypeStruct((NC, N_ACC, d_feat), _f32)]
    scratch = [
        pltpu.VMEM((CPT, CH), jnp.int32),
        pltpu.VMEM((CPT, CH), jnp.int32),
        pltpu.VMEM((CH, d_feat), _f32),
        pltpu.SemaphoreType.DMA,
        pltpu.VMEM_SHARED((N_ACC, d_feat), _f32),
    ]
    if with_e:
        outs.append(jax.ShapeDtypeStruct((NC, N_ACC, 32), _f32))
        scratch = [
            pltpu.VMEM((CPT, CH), jnp.int32),
            pltpu.VMEM((CPT, CH), jnp.int32),
            pltpu.VMEM((CH, 32), _f32),
            pltpu.VMEM((CH, d_feat), _f32),
            pltpu.SemaphoreType.DMA,
            pltpu.VMEM_SHARED((N_ACC, d_feat), _f32),
            pltpu.VMEM_SHARED((N_ACC, 32), _f32),
        ]
    return pl.kernel(functools.partial(_sc_body, with_e, d_feat),
                     out_type=outs, mesh=mesh, scratch_types=scratch)


def _enc_body(x_r, w_r, o_r):
    o_r[...] = jnp.dot(x_r[...], w_r[...], preferred_element_type=_f32)


def _enc(x, w):
    return pl.pallas_call(
        _enc_body,
        out_shape=jax.ShapeDtypeStruct((N, w.shape[1]), _f32),
    )(x, w)


def _comb_body(nout, h_r, g_r, ed_r, ap_r, aq_r, ar_r, cb_r, o_r):
    G = g_r[0, :N, :] + g_r[1, :N, :]
    Esum = ed_r[0, :N, 0:16] + ed_r[1, :N, 0:16]
    deg = ed_r[0, :N, 16:17] + ed_r[1, :N, 16:17]
    t = (deg * (jnp.dot(h_r[...], ap_r[...], preferred_element_type=_f32)
                + cb_r[0:1, :])
         + jnp.dot(G, aq_r[...], preferred_element_type=_f32)
         + jnp.dot(Esum, ar_r[...], preferred_element_type=_f32)
         + cb_r[1:2, :])
    if nout > t.shape[1]:
        t = jnp.concatenate(
            [t, jnp.zeros((N, nout - t.shape[1]), _f32)], axis=1)
    o_r[...] = t


def _comb(nout, h, g, ed, ap, aq, ar, cb):
    return pl.pallas_call(
        functools.partial(_comb_body, nout),
        out_shape=jax.ShapeDtypeStruct((N, nout), _f32),
    )(h, g, ed, ap, aq, ar, cb)


def _final_body(tp_r, z_r, ed_r, ar_r, i_r, c0_r, o_r):
    Esum = ed_r[0, :N, 0:16] + ed_r[1, :N, 0:16]
    deg = ed_r[0, :N, 16:17] + ed_r[1, :N, 16:17]
    Zs = z_r[0, :N, 0:1] + z_r[1, :N, 0:1]
    u = tp_r[:, 1:2]
    z = deg * u + Zs + jnp.dot(Esum, ar_r[...], preferred_element_type=_f32)
    gid = lax.broadcasted_iota(jnp.int32, (N, NG), 1)
    mask = (i_r[...] == gid).astype(_f32)
    sums = jnp.sum(mask * z, axis=0, keepdims=True)
    cnt = jnp.sum(mask, axis=0, keepdims=True)
    o_r[...] = sums / jnp.maximum(cnt, 1.0) + c0_r[0:1, 0:1]


def _final(tp, zp, ed, ar2, i2, c0):
    return pl.pallas_call(
        _final_body,
        out_shape=jax.ShapeDtypeStruct((1, NG), _f32),
    )(tp, zp, ed, ar2, i2, c0)


def kernel(x, edge_index, e, i, params):
    p = params

    W_dec = p['dW1'] @ p['dW2'] @ p['Wo']
    b_const = p['db1'] @ p['dW2'] @ p['Wo'] + p['db2'] @ p['Wo'] + p['bo']
    We, be = p['We'], p['be']

    fold = []
    for l in range(3):
        U = p[f'mp{l}_uW1'] @ p[f'mp{l}_uW2']
        ub = p[f'mp{l}_ub1'] @ p[f'mp{l}_uW2'] + p[f'mp{l}_ub2']
        M = p[f'mp{l}_mW2'] @ U
        W1 = p[f'mp{l}_mW1']
        ArE = W1[128:192] @ M
        fold.append((W1[0:64] @ M, W1[64:128] @ M, We @ ArE,
                     be @ ArE + p[f'mp{l}_mb1'] @ M + p[f'mp{l}_mb2'] @ U,
                     ub))

    Ap2, Aq2, Ar2, cv2, ub2 = fold[2]
    ap2 = Ap2 @ W_dec
    aq2 = Aq2 @ W_dec
    ar2 = Ar2 @ W_dec
    cv2s = cv2 @ W_dec
    ub2s = ub2 @ W_dec
    B = jnp.concatenate([aq2, ap2], axis=1)
    Ap1, Aq1, Ar1, cv1, ub1 = fold[1]
    Ap1n, Aq1n, Ar1n = Ap1 @ B, Aq1 @ B, Ar1 @ B
    cv1n, ub1n = cv1 @ B, ub1 @ B
    Ap0, Aq0, Ar0, cv0, ub0 = fold[0]

    cb0 = jnp.stack([cv0, ub0])
    cb1 = jnp.stack([cv1n, ub1n + jnp.concatenate([jnp.zeros((1,)), cv2s])])
    c0 = (ub2s + b_const).reshape(1, 1)

    src = jnp.concatenate(
        [edge_index[:, 0], jnp.zeros((E_PAD - E,), jnp.int32)])
    tgt = jnp.concatenate(
        [edge_index[:, 1], jnp.full((E_PAD - E,), DUMMY, jnp.int32)])
    src2 = src.reshape(NW * CPT, CH)
    tgt2 = tgt.reshape(NW * CPT, CH)
    e_aug = jnp.concatenate(
        [e, jnp.ones((E, 1), _f32), jnp.zeros((E, 15), _f32)], axis=1)
    e3 = jnp.concatenate(
        [e_aug, jnp.zeros((E_PAD - E, 32), _f32)]).reshape(NW * CPT, CH, 32)

    z64 = jnp.zeros((N_ACC, 64), _f32)
    z32 = jnp.zeros((N_ACC, 32), _f32)
    z16 = jnp.zeros((N_ACC, 16), _f32)
    i2 = i.reshape(N, 1)

    h0 = _enc(x, p['Wx']) + p['bx'][None, :]
    g0, ed = _make_sc(True, 64)(src2, tgt2, e3, h0, z64, z32)
    h1 = _comb(64, h0, g0, ed, Ap0, Aq0, Ar0, cb0)
    g1 = _make_sc(False, 64)(src2, tgt2, h1, z64)
    tp = _comb(16, h1, g1, ed, Ap1n, Aq1n, Ar1n, cb1)
    zp = _make_sc(False, 16)(src2, tgt2, tp, z16)
    out = _final(tp, zp, ed, ar2, i2, c0)
    return out.reshape(NG, 1)


# speedup vs baseline: 5.3521x; 5.3521x over previous
import functools

import jax
import jax.numpy as jnp
from jax import lax
from jax.experimental import pallas as pl
from jax.experimental.pallas import tpu as pltpu
from jax.experimental.pallas import tpu_sc as plsc

N = 10000
E = 320000
NG = 16
NC, NS = 2, 16
NW = NC * NS
CH = 128
CPT = 80
E_PAD = NW * CPT * CH
N_ACC = 10112
STRIPE = N_ACC // NS
DUMMY = 10008

_f32 = jnp.float32


def _sc_body(with_e, d_feat, src_r, tgt_r, *rest):
    if with_e:
        (e3_r, tab_r, zg_r, ze_r, g_out, ed_out,
         idx_s, idx_t, erows, hrows, sem, acc_g, acc_e) = rest
    else:
        (tab_r, zg_r, g_out,
         idx_s, idx_t, hrows, sem, acc_g) = rest

    c = lax.axis_index("c")
    s = lax.axis_index("s")
    wid = s * NC + c

    row0 = s * STRIPE
    pltpu.sync_copy(zg_r.at[pl.ds(row0, STRIPE)], acc_g.at[pl.ds(row0, STRIPE)])
    if with_e:
        pltpu.sync_copy(ze_r.at[pl.ds(row0, STRIPE)],
                        acc_e.at[pl.ds(row0, STRIPE)])

    pltpu.sync_copy(src_r.at[pl.ds(wid * CPT, CPT)], idx_s)
    pltpu.sync_copy(tgt_r.at[pl.ds(wid * CPT, CPT)], idx_t)
    plsc.subcore_barrier()

    def chunk(j, carry):
        if with_e:
            pltpu.sync_copy(e3_r.at[wid * CPT + j], erows)
        pltpu.async_copy(tab_r.at[idx_s.at[j]], hrows, sem).wait()
        if with_e:
            pltpu.sync_copy(erows, acc_e.at[idx_t.at[j]], add=True)
        pltpu.sync_copy(hrows, acc_g.at[idx_t.at[j]], add=True)
        return carry

    lax.fori_loop(0, CPT, chunk, 0)
    plsc.subcore_barrier()

    pltpu.sync_copy(acc_g.at[pl.ds(row0, STRIPE)],
                    g_out.at[c].at[pl.ds(row0, STRIPE)])
    if with_e:
        pltpu.sync_copy(acc_e.at[pl.ds(row0, STRIPE)],
                        ed_out.at[c].at[pl.ds(row0, STRIPE)])


def _make_sc(with_e, d_feat):
    mesh = plsc.VectorSubcoreMesh(core_axis_name="c", subcore_axis_name="s",
                                  num_cores=NC, num_subcores=NS)
    outs = [jax.ShapeDtypeStruct((NC, N_ACC, d_feat), _f32)]
    scratch = [
        pltpu.VMEM((CPT, CH), jnp.int32),
        pltpu.VMEM((CPT, CH), jnp.int32),
        pltpu.VMEM((CH, d_feat), _f32),
        pltpu.SemaphoreType.DMA,
        pltpu.VMEM_SHARED((N_ACC, d_feat), _f32),
    ]
    if with_e:
        outs.append(jax.ShapeDtypeStruct((NC, N_ACC, 32), _f32))
        scratch = [
            pltpu.VMEM((CPT, CH), jnp.int32),
            pltpu.VMEM((CPT, CH), jnp.int32),
            pltpu.VMEM((CH, 32), _f32),
            pltpu.VMEM((CH, d_feat), _f32),
            pltpu.SemaphoreType.DMA,
            pltpu.VMEM_SHARED((N_ACC, d_feat), _f32),
            pltpu.VMEM_SHARED((N_ACC, 32), _f32),
        ]
    return pl.kernel(functools.partial(_sc_body, with_e, d_feat),
                     out_type=outs, mesh=mesh, scratch_types=scratch,
                     compiler_params=pltpu.CompilerParams(
                         use_tc_tiling_on_sc=False))


def _enc_body(x_r, w_r, b_r, o_r):
    o_r[...] = (jnp.dot(x_r[...], w_r[...], preferred_element_type=_f32)
                + b_r[0:1, :])


def _enc(x, w, b):
    return pl.pallas_call(
        _enc_body,
        out_shape=jax.ShapeDtypeStruct((N, w.shape[1]), _f32),
    )(x, w, b.reshape(1, -1))


def _comb_body(nout, h_r, g_r, ed_r, ap_r, aq_r, ar_r, cb_r, o_r):
    G = g_r[0, :N, :] + g_r[1, :N, :]
    Esum = ed_r[0, :N, 0:16] + ed_r[1, :N, 0:16]
    deg = ed_r[0, :N, 16:17] + ed_r[1, :N, 16:17]
    t = (deg * (jnp.dot(h_r[...], ap_r[...], preferred_element_type=_f32)
                + cb_r[0:1, :])
         + jnp.dot(G, aq_r[...], preferred_element_type=_f32)
         + jnp.dot(Esum, ar_r[...], preferred_element_type=_f32)
         + cb_r[1:2, :])
    if nout > t.shape[1]:
        t = jnp.concatenate(
            [t, jnp.zeros((N, nout - t.shape[1]), _f32)], axis=1)
    o_r[...] = t


def _comb(nout, h, g, ed, ap, aq, ar, cb):
    return pl.pallas_call(
        functools.partial(_comb_body, nout),
        out_shape=jax.ShapeDtypeStruct((N, nout), _f32),
    )(h, g, ed, ap, aq, ar, cb)


def _final_body(tp_r, z_r, ed_r, ar_r, i_r, c0_r, o_r):
    Esum = ed_r[0, :N, 0:16] + ed_r[1, :N, 0:16]
    deg = ed_r[0, :N, 16:17] + ed_r[1, :N, 16:17]
    Zs = z_r[0, :N, 0:1] + z_r[1, :N, 0:1]
    u = tp_r[:, 1:2]
    z = deg * u + Zs + jnp.dot(Esum, ar_r[...], preferred_element_type=_f32)
    gid = lax.broadcasted_iota(jnp.int32, (N, NG), 1)
    mask = (i_r[...] == gid).astype(_f32)
    sums = jnp.sum(mask * z, axis=0, keepdims=True)
    cnt = jnp.sum(mask, axis=0, keepdims=True)
    o_r[...] = sums / jnp.maximum(cnt, 1.0) + c0_r[0:1, 0:1]


def _final(tp, zp, ed, ar2, i2, c0):
    return pl.pallas_call(
        _final_body,
        out_shape=jax.ShapeDtypeStruct((1, NG), _f32),
    )(tp, zp, ed, ar2, i2, c0)


def kernel(x, edge_index, e, i, params):
    p = params

    W_dec = p['dW1'] @ p['dW2'] @ p['Wo']
    b_const = p['db1'] @ p['dW2'] @ p['Wo'] + p['db2'] @ p['Wo'] + p['bo']
    We, be = p['We'], p['be']

    fold = []
    for l in range(3):
        U = p[f'mp{l}_uW1'] @ p[f'mp{l}_uW2']
        ub = p[f'mp{l}_ub1'] @ p[f'mp{l}_uW2'] + p[f'mp{l}_ub2']
        M = p[f'mp{l}_mW2'] @ U
        W1 = p[f'mp{l}_mW1']
        ArE = W1[128:192] @ M
        fold.append((W1[0:64] @ M, W1[64:128] @ M, We @ ArE,
                     be @ ArE + p[f'mp{l}_mb1'] @ M + p[f'mp{l}_mb2'] @ U,
                     ub))

    Ap2, Aq2, Ar2, cv2, ub2 = fold[2]
    ap2 = Ap2 @ W_dec
    aq2 = Aq2 @ W_dec
    ar2 = Ar2 @ W_dec
    cv2s = cv2 @ W_dec
    ub2s = ub2 @ W_dec
    B = jnp.concatenate([aq2, ap2], axis=1)
    Ap1, Aq1, Ar1, cv1, ub1 = fold[1]
    Ap1n, Aq1n, Ar1n = Ap1 @ B, Aq1 @ B, Ar1 @ B
    cv1n, ub1n = cv1 @ B, ub1 @ B
    Ap0, Aq0, Ar0, cv0, ub0 = fold[0]

    cb0 = jnp.stack([cv0, ub0])
    cb1 = jnp.stack([cv1n, ub1n + jnp.concatenate([jnp.zeros((1,)), cv2s])])
    c0 = (ub2s + b_const).reshape(1, 1)

    src = jnp.concatenate(
        [edge_index[:, 0], jnp.zeros((E_PAD - E,), jnp.int32)])
    tgt = jnp.concatenate(
        [edge_index[:, 1], jnp.full((E_PAD - E,), DUMMY, jnp.int32)])
    src2 = src.reshape(NW * CPT, CH)
    tgt2 = tgt.reshape(NW * CPT, CH)
    e_aug = jnp.concatenate(
        [e, jnp.ones((E, 1), _f32), jnp.zeros((E, 15), _f32)], axis=1)
    e3 = jnp.concatenate(
        [e_aug, jnp.zeros((E_PAD - E, 32), _f32)]).reshape(NW * CPT, CH, 32)

    z64 = jnp.zeros((N_ACC, 64), _f32)
    z32 = jnp.zeros((N_ACC, 32), _f32)
    z16 = jnp.zeros((N_ACC, 16), _f32)
    i2 = i.reshape(N, 1)

    h0 = _enc(x, p['Wx'], p['bx'])
    g0, ed = _make_sc(True, 64)(src2, tgt2, e3, h0, z64, z32)
    h1 = _comb(64, h0, g0, ed, Ap0, Aq0, Ar0, cb0)
    g1, = _make_sc(False, 64)(src2, tgt2, h1, z64)
    tp = _comb(16, h1, g1, ed, Ap1n, Aq1n, Ar1n, cb1)
    zp, = _make_sc(False, 16)(src2, tgt2, tp, z16)
    out = _final(tp, zp, ed, ar2, i2, c0)
    return out.reshape(NG, 1)


# speedup vs baseline: 7.8628x; 1.4691x over previous
import functools

import jax
import jax.numpy as jnp
from jax import lax
from jax.experimental import pallas as pl
from jax.experimental.pallas import tpu as pltpu
from jax.experimental.pallas import tpu_sc as plsc

N = 10000
E = 320000
NG = 16
NC, NS = 2, 16
NW = NC * NS
CH = 80
CPT = 128
NCHUNK = NW * CPT
E_PAD = NCHUNK * CH
N_ACC = 10112
STRIPE = N_ACC // NS
DUMMY = 10008

_f32 = jnp.float32


RING = 4
LOOK = 2
EC = E // CH


def _sc_body(with_e, d_feat, idx_r, tgt_r, *rest):
    if with_e:
        (tgtp_r, e3_r, ones_r, tab_r, zg_r, zed_r, g_out, ed_out,
         idx_s, idx_t, idx_tp, hb, eb, ones_b, *sems) = rest[:-2]
        acc_g, acc_ed = rest[-2:]
    else:
        (tab_r, zg_r, g_out, idx_s, idx_t, hb, *sems) = rest[:-1]
        acc_g = rest[-1]
    sg, ss = sems[:RING], sems[RING:]

    c = lax.axis_index("c")
    s = lax.axis_index("s")
    wid = s * NC + c
    base = wid * CPT

    row0 = s * STRIPE
    pltpu.sync_copy(zg_r.at[pl.ds(row0, STRIPE)], acc_g.at[pl.ds(row0, STRIPE)])
    if with_e:
        pltpu.sync_copy(zed_r.at[pl.ds(2 * row0, 2 * STRIPE)],
                        acc_ed.at[pl.ds(2 * row0, 2 * STRIPE)])
        pltpu.sync_copy(ones_r, ones_b)

    pltpu.sync_copy(idx_r.at[pl.ds(base, CPT)], idx_s)
    pltpu.sync_copy(tgt_r.at[pl.ds(base, CPT)], idx_t)
    if with_e:
        pltpu.sync_copy(tgtp_r.at[pl.ds(base, CPT)], idx_tp)
    plsc.subcore_barrier()

    def _echunk(j):
        return jnp.minimum(base + j, EC - 1)

    def _start_gathers(j, b):
        pltpu.async_copy(tab_r.at[idx_s.at[j]], hb.at[b], sg[b])
        if with_e:
            pltpu.async_copy(e3_r.at[_echunk(j)], eb.at[b], sg[b])

    def _wait_gathers(j, b):
        pltpu.make_async_copy(tab_r.at[idx_s.at[j]], hb.at[b],
                              sg[b]).wait()
        if with_e:
            pltpu.make_async_copy(e3_r.at[_echunk(j)], eb.at[b], sg[b]).wait()

    def _start_scatters(j, b):
        pltpu.async_copy(hb.at[b], acc_g.at[idx_t.at[j]], ss[b], add=True)
        if with_e:
            pltpu.async_copy(eb.at[b], acc_ed.at[idx_t.at[j]], ss[b],
                             add=True)
            pltpu.async_copy(ones_b, acc_ed.at[idx_tp.at[j]], ss[b],
                             add=True)

    def _wait_scatters(j, b):
        pltpu.make_async_copy(hb.at[b], acc_g.at[idx_t.at[j]],
                              ss[b]).wait()
        if with_e:
            pltpu.make_async_copy(eb.at[b], acc_ed.at[idx_t.at[j]],
                                  ss[b]).wait()
            pltpu.make_async_copy(ones_b, acc_ed.at[idx_tp.at[j]],
                                  ss[b]).wait()

    for b in range(LOOK):
        _start_gathers(b, b)

    def group(g, carry):
        for b in range(RING):
            j = g * RING + b
            _wait_gathers(j, b)
            _start_scatters(j, b)
            bp = (b + LOOK) % RING

            @pl.when(j >= LOOK)
            def _():
                _wait_scatters(j - LOOK, bp)

            @pl.when(j + LOOK < CPT)
            def _():
                _start_gathers(j + LOOK, bp)
        return carry

    lax.fori_loop(0, CPT // RING, group, 0)
    for b in range(LOOK):
        _wait_scatters(CPT - LOOK + b, (CPT - LOOK + b) % RING)
    plsc.subcore_barrier()

    pltpu.sync_copy(acc_g.at[pl.ds(row0, STRIPE)],
                    g_out.at[c].at[pl.ds(row0, STRIPE)])
    if with_e:
        pltpu.sync_copy(acc_ed.at[pl.ds(2 * row0, 2 * STRIPE)],
                        ed_out.at[c].at[pl.ds(2 * row0, 2 * STRIPE)])


def _make_sc(with_e, d_feat):
    mesh = plsc.VectorSubcoreMesh(core_axis_name="c", subcore_axis_name="s",
                                  num_cores=NC, num_subcores=NS)
    sems = [pltpu.SemaphoreType.DMA] * (2 * RING)
    outs = [jax.ShapeDtypeStruct((NC, N_ACC, d_feat), _f32)]
    if with_e:
        outs += [jax.ShapeDtypeStruct((NC, 2 * N_ACC, 16), _f32)]
        scratch = [
            pltpu.VMEM((CPT, CH), jnp.int32),
            pltpu.VMEM((CPT, CH), jnp.int32),
            pltpu.VMEM((CPT, CH), jnp.int32),
            pltpu.VMEM((RING, CH, d_feat), _f32),
            pltpu.VMEM((RING, CH, 16), _f32),
            pltpu.VMEM((CH, 16), _f32),
            *sems,
            pltpu.VMEM_SHARED((N_ACC, d_feat), _f32),
            pltpu.VMEM_SHARED((2 * N_ACC, 16), _f32),
        ]
    else:
        scratch = [
            pltpu.VMEM((CPT, CH), jnp.int32),
            pltpu.VMEM((CPT, CH), jnp.int32),
            pltpu.VMEM((RING, CH, d_feat), _f32),
            *sems,
            pltpu.VMEM_SHARED((N_ACC, d_feat), _f32),
        ]
    return pl.kernel(functools.partial(_sc_body, with_e, d_feat),
                     out_type=outs, mesh=mesh, scratch_types=scratch,
                     compiler_params=pltpu.CompilerParams(
                         use_tc_tiling_on_sc=False))


BN = 1000


def _enc_body(x_r, w_r, b_r, o_r):
    o_r[...] = (jnp.dot(x_r[...], w_r[...], preferred_element_type=_f32)
                + b_r[0:1, :])


def _enc(x, w, b):
    return pl.pallas_call(
        _enc_body,
        grid=(N // BN,),
        in_specs=[
            pl.BlockSpec((BN, x.shape[1]), lambda i: (i, 0)),
            pl.BlockSpec((x.shape[1], w.shape[1]), lambda i: (0, 0)),
            pl.BlockSpec((1, w.shape[1]), lambda i: (0, 0)),
        ],
        out_specs=pl.BlockSpec((BN, w.shape[1]), lambda i: (i, 0)),
        out_shape=jax.ShapeDtypeStruct((N, w.shape[1]), _f32),
    )(x, w, b.reshape(1, -1))


def _comb_body(nout, h_r, g_r, eo_r, dd_r, ap_r, aq_r, ar_r, cb_r, o_r):
    G = g_r[0] + g_r[1]
    Esum = eo_r[0] + eo_r[1]
    deg = dd_r[0, :, 0:1] + dd_r[1, :, 0:1]
    t = (deg * (jnp.dot(h_r[...], ap_r[...], preferred_element_type=_f32)
                + cb_r[0:1, :])
         + jnp.dot(G, aq_r[...], preferred_element_type=_f32)
         + jnp.dot(Esum, ar_r[...], preferred_element_type=_f32)
         + cb_r[1:2, :])
    if nout > t.shape[1]:
        t = jnp.concatenate(
            [t, jnp.zeros((BN, nout - t.shape[1]), _f32)], axis=1)
    o_r[...] = t


def _comb(nout, h, g, eo, dd, ap, aq, ar, cb):
    return pl.pallas_call(
        functools.partial(_comb_body, nout),
        grid=(N // BN,),
        in_specs=[
            pl.BlockSpec((BN, 64), lambda i: (i, 0)),
            pl.BlockSpec((2, BN, 64), lambda i: (0, i, 0)),
            pl.BlockSpec((2, BN, 16), lambda i: (0, i, 0)),
            pl.BlockSpec((2, BN, 16), lambda i: (0, i, 0)),
            pl.BlockSpec(ap.shape, lambda i: (0, 0)),
            pl.BlockSpec(aq.shape, lambda i: (0, 0)),
            pl.BlockSpec(ar.shape, lambda i: (0, 0)),
            pl.BlockSpec(cb.shape, lambda i: (0, 0)),
        ],
        out_specs=pl.BlockSpec((BN, nout), lambda i: (i, 0)),
        out_shape=jax.ShapeDtypeStruct((N, nout), _f32),
    )(h, g, eo, dd, ap, aq, ar, cb)


def _final_body(tp_r, z_r, eo_r, dd_r, ar_r, i_r, c0_r, o_r, acc_r):
    k = pl.program_id(0)
    Esum = eo_r[0] + eo_r[1]
    deg = dd_r[0, :, 0:1] + dd_r[1, :, 0:1]
    Zs = z_r[0, :, 0:1] + z_r[1, :, 0:1]
    u = tp_r[:, 1:2]
    z = deg * u + Zs + jnp.dot(Esum, ar_r[...], preferred_element_type=_f32)
    gid = lax.broadcasted_iota(jnp.int32, (BN, NG), 1)
    mask = (i_r[...] == gid).astype(_f32)
    sums = jnp.sum(mask * z, axis=0, keepdims=True)
    cnt = jnp.sum(mask, axis=0, keepdims=True)
    part = jnp.concatenate([sums, cnt], axis=0)

    @pl.when(k == 0)
    def _():
        acc_r[...] = jnp.zeros_like(acc_r)

    acc_r[...] += part

    @pl.when(k == N // BN - 1)
    def _():
        a = acc_r[...]
        o_r[...] = a[0:1, :] / jnp.maximum(a[1:2, :], 1.0) + c0_r[0:1, 0:1]


def _final(tp, zp, eo, dd, ar2, i2, c0):
    return pl.pallas_call(
        _final_body,
        grid=(N // BN,),
        in_specs=[
            pl.BlockSpec((BN, 16), lambda i: (i, 0)),
            pl.BlockSpec((2, BN, 16), lambda i: (0, i, 0)),
            pl.BlockSpec((2, BN, 16), lambda i: (0, i, 0)),
            pl.BlockSpec((2, BN, 16), lambda i: (0, i, 0)),
            pl.BlockSpec((16, 1), lambda i: (0, 0)),
            pl.BlockSpec((BN, 1), lambda i: (i, 0)),
            pl.BlockSpec((1, 1), lambda i: (0, 0)),
        ],
        out_specs=pl.BlockSpec((1, NG), lambda i: (0, 0)),
        out_shape=jax.ShapeDtypeStruct((1, NG), _f32),
        scratch_shapes=[pltpu.VMEM((2, NG), _f32)],
    )(tp, zp, eo, dd, ar2, i2, c0)


def kernel(x, edge_index, e, i, params):
    return _kernel(x, edge_index, e, i, params)


def _kernel(x, edge_index, e, i, params):
    p = params

    W_dec = p['dW1'] @ p['dW2'] @ p['Wo']
    b_const = p['db1'] @ p['dW2'] @ p['Wo'] + p['db2'] @ p['Wo'] + p['bo']
    We, be = p['We'], p['be']

    fold = []
    for l in range(3):
        U = p[f'mp{l}_uW1'] @ p[f'mp{l}_uW2']
        ub = p[f'mp{l}_ub1'] @ p[f'mp{l}_uW2'] + p[f'mp{l}_ub2']
        M = p[f'mp{l}_mW2'] @ U
        W1 = p[f'mp{l}_mW1']
        ArE = W1[128:192] @ M
        fold.append((W1[0:64] @ M, W1[64:128] @ M, We @ ArE,
                     be @ ArE + p[f'mp{l}_mb1'] @ M + p[f'mp{l}_mb2'] @ U,
                     ub))

    Ap2, Aq2, Ar2, cv2, ub2 = fold[2]
    ap2 = Ap2 @ W_dec
    aq2 = Aq2 @ W_dec
    ar2 = Ar2 @ W_dec
    cv2s = cv2 @ W_dec
    ub2s = ub2 @ W_dec
    B = jnp.concatenate([aq2, ap2], axis=1)
    Ap1, Aq1, Ar1, cv1, ub1 = fold[1]
    Ap1n, Aq1n, Ar1n = Ap1 @ B, Aq1 @ B, Ar1 @ B
    cv1n, ub1n = cv1 @ B, ub1 @ B
    Ap0, Aq0, Ar0, cv0, ub0 = fold[0]

    cb0 = jnp.stack([cv0, ub0])
    cb1 = jnp.stack([cv1n, ub1n + jnp.concatenate([jnp.zeros((1,)), cv2s])])
    c0 = (ub2s + b_const).reshape(1, 1)

    src = jnp.concatenate(
        [edge_index[:, 0], jnp.zeros((E_PAD - E,), jnp.int32)])
    tgt = jnp.concatenate(
        [edge_index[:, 1], jnp.full((E_PAD - E,), DUMMY, jnp.int32)])
    src2 = src.reshape(NCHUNK, CH)
    tgt2 = tgt.reshape(NCHUNK, CH)
    tgtp2 = tgt2 + N_ACC
    e3 = e.reshape(EC, CH, 16)
    ones16 = jnp.zeros((CH, 16), _f32).at[:, 0].set(1.0)

    z64 = jnp.zeros((N_ACC, 64), _f32)
    z16 = jnp.zeros((N_ACC, 16), _f32)
    zed = jnp.zeros((2 * N_ACC, 16), _f32)
    i2 = i.reshape(N, 1)

    h0 = _enc(x, p['Wx'], p['bx'])
    g0, ed = _make_sc(True, 64)(src2, tgt2, tgtp2, e3, ones16, h0, z64, zed)
    edr = ed.reshape(NC, 2, N_ACC, 16)
    eo, dd = edr[:, 0], edr[:, 1]
    h1 = _comb(64, h0, g0, eo, dd, Ap0, Aq0, Ar0, cb0)
    g1, = _make_sc(False, 64)(src2, tgt2, h1, z64)
    tp = _comb(16, h1, g1, eo, dd, Ap1n, Aq1n, Ar1n, cb1)
    zp, = _make_sc(False, 16)(src2, tgt2, tp, z16)
    out = _final(tp, zp, eo, dd, ar2, i2, c0)
    return out.reshape(NG, 1)
